# Initial kernel scaffold; baseline (speedup 1.0000x reference)
import jax
import jax.numpy as jnp
from jax.experimental import pallas as pl


def kernel(RNA_supports, protein_supports, RNA_inputs, protein_inputs, W0, W1, SW0, SW1):
    raise NotImplementedError("write your pallas kernel here")


# speedup vs baseline: 1.0549x; 1.0549x over previous
import functools

import jax
import jax.numpy as jnp
from jax.experimental import pallas as pl
from jax.experimental.pallas import tpu as pltpu

N = 4096
BLOCK = 256


def _layer_kernel(sr_ref, sp_ref, hr_ref, hp_ref, w_ref, sw_ref,
                  out_r_ref, out_p_ref,
                  vu_ref, vv_ref, self_r_ref, self_p_ref, *, block, k):
    i = pl.program_id(0)

    @pl.when(i == 0)
    def _init():
        hr = hr_ref[...]
        hp = hp_ref[...]
        w0 = w_ref[0]
        w1 = w_ref[1]
        sw = sw_ref[...]
        vu_ref[...] = jnp.concatenate(
            [jnp.dot(hr, w0, preferred_element_type=jnp.float32),
             jnp.dot(hr, w1, preferred_element_type=jnp.float32)],
            axis=1).astype(jnp.bfloat16)
        vv_ref[...] = jnp.concatenate(
            [jnp.dot(hp, w0, preferred_element_type=jnp.float32),
             jnp.dot(hp, w1, preferred_element_type=jnp.float32)],
            axis=1).astype(jnp.bfloat16)
        self_r_ref[...] = jnp.dot(hr, sw, preferred_element_type=jnp.float32)
        self_p_ref[...] = jnp.dot(hp, sw, preferred_element_type=jnp.float32)

    vu = vu_ref[...]
    vv = vv_ref[...]
    sr0 = sr_ref[0].astype(jnp.bfloat16)
    sr1 = sr_ref[1].astype(jnp.bfloat16)
    sp0 = sp_ref[0].astype(jnp.bfloat16)
    sp1 = sp_ref[1].astype(jnp.bfloat16)
    rows = pl.ds(i * block, block)
    agg_r = jnp.concatenate(
        [jnp.dot(sr0, vv[:, :k], preferred_element_type=jnp.float32),
         jnp.dot(sr1, vv[:, k:], preferred_element_type=jnp.float32)],
        axis=1)
    agg_p = jnp.concatenate(
        [jnp.dot(sp0, vu[:, :k], preferred_element_type=jnp.float32),
         jnp.dot(sp1, vu[:, k:], preferred_element_type=jnp.float32)],
        axis=1)
    out_r_ref[...] = jnp.maximum(agg_r + self_r_ref[rows, :], 0.0)
    out_p_ref[...] = jnp.maximum(agg_p + self_p_ref[rows, :], 0.0)


def _gcn_layer(S_r, S_p, H_r, H_p, W, SW, *, block=BLOCK):
    d = H_r.shape[1]
    k = W.shape[2]
    nblk = N // block
    kern = functools.partial(_layer_kernel, block=block, k=k)
    out_shape = jax.ShapeDtypeStruct((N, 2 * k), jnp.float32)
    full = lambda i: (0, 0)
    return pl.pallas_call(
        kern,
        grid_spec=pltpu.PrefetchScalarGridSpec(
            num_scalar_prefetch=0,
            grid=(nblk,),
            in_specs=[
                pl.BlockSpec((2, block, N), lambda i: (0, i, 0)),
                pl.BlockSpec((2, block, N), lambda i: (0, i, 0)),
                pl.BlockSpec((N, d), full),
                pl.BlockSpec((N, d), full),
                pl.BlockSpec((2, d, k), lambda i: (0, 0, 0)),
                pl.BlockSpec((d, 2 * k), full),
            ],
            out_specs=[
                pl.BlockSpec((block, 2 * k), lambda i: (i, 0)),
                pl.BlockSpec((block, 2 * k), lambda i: (i, 0)),
            ],
            scratch_shapes=[
                pltpu.VMEM((N, 2 * k), jnp.bfloat16),
                pltpu.VMEM((N, 2 * k), jnp.bfloat16),
                pltpu.VMEM((N, 2 * k), jnp.float32),
                pltpu.VMEM((N, 2 * k), jnp.float32),
            ],
        ),
        out_shape=[out_shape, out_shape],
        compiler_params=pltpu.CompilerParams(
            dimension_semantics=("arbitrary",),
        ),
    )(S_r, S_p, H_r, H_p, W, SW)


def kernel(RNA_supports, protein_supports, RNA_inputs, protein_inputs,
           W0, W1, SW0, SW1):
    h_r, h_p = _gcn_layer(RNA_supports, protein_supports,
                          RNA_inputs, protein_inputs, W0, SW0)
    h_r, h_p = _gcn_layer(RNA_supports, protein_supports, h_r, h_p, W1, SW1)
    return (h_r, h_p)
